# baseline (speedup 1.0000x reference)
import jax
import jax.numpy as jnp
from jax import lax
from jax.experimental import pallas as pl
from jax.experimental.pallas import tpu as pltpu
from jax.experimental.pallas import tpu_sc as plsc

N_NODES = 100000
NUM_TYPES = 128
L = 16
NW = 32
NSLOTS = 5
CHUNK = 160
NFULL = N_NODES // CHUNK
MAX_ITERS = -(-NFULL // NW)

TC_BLOCK = 20000


def _scatter_groups(buf, idx_v, n_rows, value):
    vals = jnp.full((L,), value, dtype=jnp.float32)
    row_off = lax.broadcasted_iota(jnp.int32, (L,), 0) * NUM_TYPES
    for g in range(n_rows // L):
        cols = idx_v[pl.ds(g * L, L)]
        flat = row_off + (g * L * NUM_TYPES) + cols
        plsc.store_scatter(buf, [flat], vals)


def _zero_fill(buf, n_words):
    zeros = jnp.zeros((L,), jnp.float32)

    def _step(k, _):
        for j in range(8):
            buf[pl.ds(k * 8 * L + j * L, L)] = zeros
        return 0

    lax.fori_loop(0, n_words // (8 * L), _step, 0)


def _sc_body(elem_hbm, out_hbm, idx0, idx1, idx2, idx3, idx4,
             buf0, buf1, buf2, buf3, buf4, sem0, sem1, sem2, sem3, sem4):
    wid = lax.axis_index("s") * 2 + lax.axis_index("c")
    idx_s = (idx0, idx1, idx2, idx3, idx4)
    buf_s = (buf0, buf1, buf2, buf3, buf4)
    sem_s = (sem0, sem1, sem2, sem3, sem4)

    def _wait_slot(s):
        pltpu.make_async_copy(buf_s[s], out_hbm.at[pl.ds(0, CHUNK * NUM_TYPES)],
                              sem_s[s]).wait()

    def _fire(s, base):
        pltpu.sync_copy(elem_hbm.at[pl.ds(base, CHUNK)], idx_s[s])
        _scatter_groups(buf_s[s], idx_s[s], CHUNK, 1.0)
        pltpu.async_copy(buf_s[s],
                         out_hbm.at[pl.ds(base * NUM_TYPES, CHUNK * NUM_TYPES)],
                         sem_s[s])

    for i in range(NSLOTS):
        _zero_fill(buf_s[i], CHUNK * NUM_TYPES)
        _fire(i, (wid + i * NW) * CHUNK)

    def _round(r, _):
        for j in range(NSLOTS):
            _wait_slot(j)
            _scatter_groups(buf_s[j], idx_s[j], CHUNK, 0.0)
            c_i = wid + (r * NSLOTS + j) * NW

            @pl.when(c_i < NFULL)
            def _(j=j, c_i=c_i):
                _fire(j, c_i * CHUNK)

        return 0

    lax.fori_loop(1, MAX_ITERS // NSLOTS, _round, 0)

    for i in range(MAX_ITERS - NSLOTS, MAX_ITERS):
        c_i = wid + i * NW

        @pl.when(c_i < NFULL)
        def _(s=i % NSLOTS):
            _wait_slot(s)


def _tc_body(idx_ref, out_ref):
    idx = jnp.reshape(idx_ref[...], (TC_BLOCK, 1))
    classes = lax.broadcasted_iota(jnp.int32, (1, NUM_TYPES), 1)
    out_ref[...] = jnp.where(idx == classes, jnp.float32(1.0), jnp.float32(0.0))


def kernel(elem_map, pos):
    del pos
    idx_lin = jnp.reshape(elem_map, (N_NODES,))

    out_sds = jax.ShapeDtypeStruct((N_NODES * NUM_TYPES,), jnp.float32)
    mesh = plsc.VectorSubcoreMesh(core_axis_name="c", subcore_axis_name="s")
    sc_call = pl.kernel(
        _sc_body,
        out_type=out_sds,
        mesh=mesh,
        compiler_params=pltpu.CompilerParams(needs_layout_passes=False,
                                             skip_device_barrier=True),
        scratch_types=(
            [pltpu.VMEM((CHUNK,), jnp.int32)] * NSLOTS
            + [pltpu.VMEM((CHUNK * NUM_TYPES,), jnp.float32)] * NSLOTS
            + [pltpu.SemaphoreType.DMA] * NSLOTS
        ),
    )
    oh2 = sc_call(idx_lin)

    idx3d = jnp.reshape(idx_lin, (N_NODES // TC_BLOCK, 1, TC_BLOCK))
    oh1 = pl.pallas_call(
        _tc_body,
        grid=(N_NODES // TC_BLOCK,),
        in_specs=[pl.BlockSpec((1, 1, TC_BLOCK), lambda i: (i, 0, 0))],
        out_specs=pl.BlockSpec((TC_BLOCK, NUM_TYPES), lambda i: (i, 0)),
        out_shape=jax.ShapeDtypeStruct((N_NODES, NUM_TYPES), jnp.float32),
    )(idx3d)

    return (oh1, jnp.reshape(oh2, (N_NODES, NUM_TYPES)))


# speedup vs baseline: 1.0091x; 1.0091x over previous
import jax
import jax.numpy as jnp
from jax import lax
from jax.experimental import pallas as pl
from jax.experimental.pallas import tpu as pltpu
from jax.experimental.pallas import tpu_sc as plsc

N_NODES = 100000
NUM_TYPES = 128
L = 16
NW = 32
NSLOTS = 5
CHUNK = 160
NFULL = N_NODES // CHUNK
MAX_ITERS = -(-NFULL // NW)

TC_BLOCK = 10000


def _scatter_groups(buf, idx_v, n_rows, value):
    vals = jnp.full((L,), value, dtype=jnp.float32)
    row_off = lax.broadcasted_iota(jnp.int32, (L,), 0) * NUM_TYPES
    for g in range(n_rows // L):
        cols = idx_v[pl.ds(g * L, L)]
        flat = row_off + (g * L * NUM_TYPES) + cols
        plsc.store_scatter(buf, [flat], vals)


def _zero_fill(buf, n_words):
    zeros = jnp.zeros((L,), jnp.float32)

    def _step(k, _):
        for j in range(8):
            buf[pl.ds(k * 8 * L + j * L, L)] = zeros
        return 0

    lax.fori_loop(0, n_words // (8 * L), _step, 0)


def _sc_body(elem_hbm, out_hbm, idx0, idx1, idx2, idx3, idx4,
             buf0, buf1, buf2, buf3, buf4, sem0, sem1, sem2, sem3, sem4):
    wid = lax.axis_index("s") * 2 + lax.axis_index("c")
    idx_s = (idx0, idx1, idx2, idx3, idx4)
    buf_s = (buf0, buf1, buf2, buf3, buf4)
    sem_s = (sem0, sem1, sem2, sem3, sem4)

    def _wait_slot(s):
        pltpu.make_async_copy(buf_s[s], out_hbm.at[pl.ds(0, CHUNK * NUM_TYPES)],
                              sem_s[s]).wait()

    def _fire(s, base):
        pltpu.sync_copy(elem_hbm.at[pl.ds(base, CHUNK)], idx_s[s])
        _scatter_groups(buf_s[s], idx_s[s], CHUNK, 1.0)
        pltpu.async_copy(buf_s[s],
                         out_hbm.at[pl.ds(base * NUM_TYPES, CHUNK * NUM_TYPES)],
                         sem_s[s])

    for i in range(NSLOTS):
        _zero_fill(buf_s[i], CHUNK * NUM_TYPES)
        _fire(i, (wid + i * NW) * CHUNK)

    def _round(r, _):
        for j in range(NSLOTS):
            _wait_slot(j)
            _scatter_groups(buf_s[j], idx_s[j], CHUNK, 0.0)
            c_i = wid + (r * NSLOTS + j) * NW

            @pl.when(c_i < NFULL)
            def _(j=j, c_i=c_i):
                _fire(j, c_i * CHUNK)

        return 0

    lax.fori_loop(1, MAX_ITERS // NSLOTS, _round, 0)

    for i in range(MAX_ITERS - NSLOTS, MAX_ITERS):
        c_i = wid + i * NW

        @pl.when(c_i < NFULL)
        def _(s=i % NSLOTS):
            _wait_slot(s)


def _tc_body(idx_ref, out_ref):
    idx = jnp.reshape(idx_ref[...], (TC_BLOCK, 1))
    classes = lax.broadcasted_iota(jnp.int32, (1, NUM_TYPES), 1)
    out_ref[...] = jnp.where(idx == classes, jnp.float32(1.0), jnp.float32(0.0))


def kernel(elem_map, pos):
    del pos
    idx_lin = jnp.reshape(elem_map, (N_NODES,))

    out_sds = jax.ShapeDtypeStruct((N_NODES * NUM_TYPES,), jnp.float32)
    mesh = plsc.VectorSubcoreMesh(core_axis_name="c", subcore_axis_name="s")
    sc_call = pl.kernel(
        _sc_body,
        out_type=out_sds,
        mesh=mesh,
        compiler_params=pltpu.CompilerParams(needs_layout_passes=False),
        scratch_types=(
            [pltpu.VMEM((CHUNK,), jnp.int32)] * NSLOTS
            + [pltpu.VMEM((CHUNK * NUM_TYPES,), jnp.float32)] * NSLOTS
            + [pltpu.SemaphoreType.DMA] * NSLOTS
        ),
    )
    oh2 = sc_call(idx_lin)

    idx3d = jnp.reshape(idx_lin, (N_NODES // TC_BLOCK, 1, TC_BLOCK))
    oh1 = pl.pallas_call(
        _tc_body,
        grid=(N_NODES // TC_BLOCK,),
        in_specs=[pl.BlockSpec((1, 1, TC_BLOCK), lambda i: (i, 0, 0))],
        out_specs=pl.BlockSpec((TC_BLOCK, NUM_TYPES), lambda i: (i, 0)),
        out_shape=jax.ShapeDtypeStruct((N_NODES, NUM_TYPES), jnp.float32),
    )(idx3d)

    return (oh1, jnp.reshape(oh2, (N_NODES, NUM_TYPES)))
